# Initial kernel scaffold; baseline (speedup 1.0000x reference)
import jax
import jax.numpy as jnp
from jax.experimental import pallas as pl


def kernel(x, hx_0, hx_1, tree_ids_d, tree_ids_dr, tree_ids_dl, W_ioux, W_iouh0, b_iouh0, W_iouh1, b_iouh1, W_fx, W_fh0, b_fh0, W_fh1, b_fh1, W_fh2, b_fh2, W_fh3, b_fh3):
    raise NotImplementedError("write your pallas kernel here")


# speedup vs baseline: 146.6848x; 146.6848x over previous
import functools

import jax
import jax.numpy as jnp
from jax import lax
from jax.experimental import pallas as pl
from jax.experimental.pallas import tpu as pltpu


def _stage_a_body(x_ref, hx0_ref, hx1_ref, iddr_ref, idrr_ref, idlr_ref,
                  iddc_ref, idrc_ref, idlc_ref,
                  wx_ref, wh_ref, bh_ref, h_ref, c_ref, *, L, H):
    f32 = jnp.float32
    xb = x_ref[0]
    h0 = hx0_ref[0]
    h1 = hx1_ref[0]
    xw = jnp.dot(xb, wx_ref[...], preferred_element_type=f32)
    hw = jnp.dot(h0, wh_ref[...], preferred_element_type=f32) + bh_ref[...]
    tid_d_r = iddr_ref[0]
    tid_r_r = idrr_ref[0]
    tid_l_r = idlr_ref[0]
    tid_d_c = iddc_ref[0]
    tid_r_c = idrc_ref[0]
    tid_l_c = idlc_ref[0]
    row = lax.broadcasted_iota(jnp.int32, (L, L), 0)
    col = lax.broadcasted_iota(jnp.int32, (L, L), 1)
    prt = (row == tid_r_r).astype(f32)
    plt = (row == tid_l_r).astype(f32)
    s = jnp.dot(prt, hw[:, :H], preferred_element_type=f32)
    s = s + jnp.dot(plt, hw[:, H:2 * H], preferred_element_type=f32)
    pd = (tid_d_c == col).astype(f32)
    pr = (tid_r_c == col).astype(f32)
    pl_ = (tid_l_c == col).astype(f32)
    fpre = jnp.dot(pd, xw[:, 3 * H:], preferred_element_type=f32)
    fpre = fpre + jnp.dot(pr, hw[:, 2 * H:3 * H], preferred_element_type=f32)
    fpre = fpre + jnp.dot(pl_, hw[:, 3 * H:], preferred_element_type=f32)
    i = jax.nn.sigmoid(xw[:, :H] + s)
    o = jax.nn.sigmoid(xw[:, H:2 * H])
    u = jnp.tanh(xw[:, 2 * H:3 * H])
    f = jax.nn.sigmoid(fpre)
    fc = f * h1
    pdt = (row == tid_d_r).astype(f32)
    c = i * u + jnp.dot(pdt, fc, preferred_element_type=f32)
    h_ref[0] = o * jnp.tanh(c)
    c_ref[0] = c


def _stage_b_body(hflat_ref, cflat_ref, hx0_ref, hx1_ref, iddc_ref,
                  hout_ref, cout_ref, bufh, bufc, semh, semc, base_ref,
                  *, L, H, BL):
    f32 = jnp.float32
    W = L + 16
    b = pl.program_id(0)

    @pl.when(b == 0)
    def _():
        base_ref[0] = 0

    base = base_ref[0]
    base8 = jnp.minimum((base // 8) * 8, BL - W)
    base8 = pl.multiple_of(base8, 8)
    off = base - base8
    cp1 = pltpu.make_async_copy(hflat_ref.at[pl.ds(base8, W)], bufh, semh)
    cp2 = pltpu.make_async_copy(cflat_ref.at[pl.ds(base8, W)], bufc, semc)
    cp1.start()
    cp2.start()
    tid_d = iddc_ref[0]
    mask = tid_d != 0
    mf = mask.astype(f32)
    row = lax.broadcasted_iota(jnp.int32, (L, L), 0)
    col = lax.broadcasted_iota(jnp.int32, (L, L), 1)
    tri = (col <= row).astype(f32)
    rank_inc = jnp.dot(tri, mf, preferred_element_type=f32)
    rank = rank_inc.astype(jnp.int32) - 1 + off
    colw = lax.broadcasted_iota(jnp.int32, (L, W), 1)
    p = ((rank == colw) & mask).astype(f32)
    cp1.wait()
    cp2.wait()
    gh = jnp.dot(p, bufh[...], preferred_element_type=f32)
    gc = jnp.dot(p, bufc[...], preferred_element_type=f32)
    hout_ref[0] = jnp.where(mask, gh, hx0_ref[0])
    cout_ref[0] = jnp.where(mask, gc, hx1_ref[0])
    base_ref[0] = base + jnp.sum(mask.astype(jnp.int32))


def kernel(x, hx_0, hx_1, tree_ids_d, tree_ids_dr, tree_ids_dl,
           W_ioux, W_iouh0, b_iouh0, W_iouh1, b_iouh1, W_fx,
           W_fh0, b_fh0, W_fh1, b_fh1, W_fh2, b_fh2, W_fh3, b_fh3):
    B, L, E = x.shape
    H = W_fx.shape[1]
    f32 = jnp.float32
    wx_all = jnp.concatenate([W_ioux, W_fx], axis=1)
    wh_all = jnp.concatenate([W_iouh0[:, :H], W_iouh1[:, :H],
                              W_fh0 + W_fh1, W_fh2 + W_fh3], axis=1)
    bh_all = jnp.concatenate([b_iouh0[:H], b_iouh1[:H],
                              b_fh0 + b_fh1, b_fh2 + b_fh3]).reshape(1, 4 * H)
    idd_r = tree_ids_d.reshape(B, 1, L)
    idr_r = tree_ids_dr.reshape(B, 1, L)
    idl_r = tree_ids_dl.reshape(B, 1, L)
    idd_c = tree_ids_d.reshape(B, L, 1)
    idr_c = tree_ids_dr.reshape(B, L, 1)
    idl_c = tree_ids_dl.reshape(B, L, 1)

    row_spec = pl.BlockSpec((1, 1, L), lambda b: (b, 0, 0))
    col_spec = pl.BlockSpec((1, L, 1), lambda b: (b, 0, 0))
    bh_spec = pl.BlockSpec((1, L, H), lambda b: (b, 0, 0))

    h_full, c_full = pl.pallas_call(
        functools.partial(_stage_a_body, L=L, H=H),
        grid=(B,),
        in_specs=[
            pl.BlockSpec((1, L, E), lambda b: (b, 0, 0)),
            bh_spec, bh_spec,
            row_spec, row_spec, row_spec,
            col_spec, col_spec, col_spec,
            pl.BlockSpec((E, 4 * H), lambda b: (0, 0)),
            pl.BlockSpec((H, 4 * H), lambda b: (0, 0)),
            pl.BlockSpec((1, 4 * H), lambda b: (0, 0)),
        ],
        out_specs=[bh_spec, bh_spec],
        out_shape=[
            jax.ShapeDtypeStruct((B, L, H), f32),
            jax.ShapeDtypeStruct((B, L, H), f32),
        ],
        compiler_params=pltpu.CompilerParams(
            dimension_semantics=("arbitrary",)),
    )(x, hx_0, hx_1, idd_r, idr_r, idl_r, idd_c, idr_c, idl_c,
      wx_all, wh_all, bh_all)

    hflat = h_full.reshape(B * L, H)
    cflat = c_full.reshape(B * L, H)

    h_out, c_out = pl.pallas_call(
        functools.partial(_stage_b_body, L=L, H=H, BL=B * L),
        grid=(B,),
        in_specs=[
            pl.BlockSpec(memory_space=pl.ANY),
            pl.BlockSpec(memory_space=pl.ANY),
            bh_spec, bh_spec,
            col_spec,
        ],
        out_specs=[bh_spec, bh_spec],
        out_shape=[
            jax.ShapeDtypeStruct((B, L, H), f32),
            jax.ShapeDtypeStruct((B, L, H), f32),
        ],
        scratch_shapes=[
            pltpu.VMEM((L + 16, H), f32),
            pltpu.VMEM((L + 16, H), f32),
            pltpu.SemaphoreType.DMA,
            pltpu.SemaphoreType.DMA,
            pltpu.SMEM((1,), jnp.int32),
        ],
        compiler_params=pltpu.CompilerParams(
            dimension_semantics=("arbitrary",)),
    )(hflat, cflat, hx_0, hx_1, idd_c)

    return (h_out, c_out)
